# Initial kernel scaffold; baseline (speedup 1.0000x reference)
import functools

import jax
import jax.numpy as jnp
from jax import lax
from jax.experimental import pallas as pl
from jax.experimental.pallas import tpu as pltpu
from jax.experimental.pallas import tpu_sc as plsc

_NC = 2
_NS = 16
_NW = _NC * _NS
_L = 16
_CHUNK = 128
_EPS = 1e-5


def _sc_mesh():
    return plsc.VectorSubcoreMesh(
        core_axis_name="c", subcore_axis_name="s",
        num_cores=_NC, num_subcores=_NS)


def _sc_degree(edge_index, n_pad, n_chunks):
    rpt = n_pad // _NS

    @functools.partial(
        pl.kernel,
        out_type=jax.ShapeDtypeStruct((_NC, n_pad, _L), jnp.float32),
        mesh=_sc_mesh(),
        scratch_types=[
            pltpu.VMEM((_CHUNK,), jnp.int32),
            pltpu.VMEM((_CHUNK, _L), jnp.float32),
            pltpu.VMEM((_CHUNK, _L), jnp.float32),
            pltpu.VMEM_SHARED((n_pad, _L), jnp.float32),
        ],
    )
    def k(ei_hbm, out_hbm, idx_v, ones_v, zero_v, acc):
        cid = lax.axis_index("c")
        sid = lax.axis_index("s")
        w = sid * _NC + cid

        @pl.loop(0, _CHUNK)
        def _(r):
            ones_v[r] = jnp.ones((_L,), jnp.float32)
            zero_v[r] = jnp.zeros((_L,), jnp.float32)

        @pl.loop(0, rpt, step=_CHUNK)
        def _(r0):
            pltpu.sync_copy(zero_v, acc.at[pl.ds(sid * rpt + r0, _CHUNK)])

        plsc.subcore_barrier()

        @pl.loop(w, n_chunks, step=_NW)
        def _(c):
            pltpu.sync_copy(ei_hbm.at[1, pl.ds(c * _CHUNK, _CHUNK)], idx_v)
            pltpu.sync_copy(ones_v, acc.at[idx_v], add=True)

        plsc.subcore_barrier()
        pltpu.sync_copy(acc.at[pl.ds(sid * rpt, rpt)],
                        out_hbm.at[cid, pl.ds(sid * rpt, rpt)])

    return k(edge_index)


def _sc_scatter(h, edge_index, n_pad, n_chunks):
    d = h.shape[1]
    rpt = n_pad // _NS

    @functools.partial(
        pl.kernel,
        out_type=jax.ShapeDtypeStruct((_NC, n_pad, d), jnp.float32),
        mesh=_sc_mesh(),
        scratch_types=[
            pltpu.VMEM((_CHUNK,), jnp.int32),
            pltpu.VMEM((_CHUNK,), jnp.int32),
            pltpu.VMEM((_CHUNK, d), jnp.float32),
            pltpu.VMEM_SHARED((n_pad, d), jnp.float32),
            pltpu.SemaphoreType.DMA,
        ],
    )
    def k(h_hbm, ei_hbm, out_hbm, src_v, dst_v, rows_v, acc, sem):
        cid = lax.axis_index("c")
        sid = lax.axis_index("s")
        w = sid * _NC + cid

        @pl.loop(0, _CHUNK)
        def _(r):
            for c0 in range(0, d, _L):
                rows_v[r, pl.ds(c0, _L)] = jnp.zeros((_L,), jnp.float32)

        @pl.loop(0, rpt, step=_CHUNK)
        def _(r0):
            pltpu.sync_copy(rows_v, acc.at[pl.ds(sid * rpt + r0, _CHUNK)])

        plsc.subcore_barrier()

        @pl.loop(w, n_chunks, step=_NW)
        def _(c):
            e0 = c * _CHUNK
            pltpu.sync_copy(ei_hbm.at[0, pl.ds(e0, _CHUNK)], src_v)
            pltpu.sync_copy(ei_hbm.at[1, pl.ds(e0, _CHUNK)], dst_v)
            pltpu.async_copy(h_hbm.at[src_v], rows_v, sem).wait()
            pltpu.sync_copy(rows_v, acc.at[dst_v], add=True)

        plsc.subcore_barrier()
        pltpu.sync_copy(acc.at[pl.ds(sid * rpt, rpt)],
                        out_hbm.at[cid, pl.ds(sid * rpt, rpt)])

    return k(h, edge_index)


def _pick_br(n):
    for br in (512, 400, 256, 200, 128, 80, 40, 16, 8):
        if n % br == 0:
            return br
    return n


def _dinv_block(hp_ref):
    deg = hp_ref[0, :, 0:1] + hp_ref[1, :, 0:1] + 1.0
    return lax.rsqrt(deg)


def _mm_body(x_ref, w_ref, hp_ref, o_ref):
    dv = _dinv_block(hp_ref)
    o_ref[...] = jnp.dot(x_ref[...], w_ref[...],
                         preferred_element_type=jnp.float32) * dv


def _tc_matmul_scale(x, w, hp):
    n, d = x.shape
    br = _pick_br(n)
    return pl.pallas_call(
        _mm_body,
        grid=(n // br,),
        in_specs=[
            pl.BlockSpec((br, d), lambda i: (i, 0)),
            pl.BlockSpec((d, d), lambda i: (0, 0)),
            pl.BlockSpec((2, br, _L), lambda i: (0, i, 0)),
        ],
        out_specs=pl.BlockSpec((br, d), lambda i: (i, 0)),
        out_shape=jax.ShapeDtypeStruct((n, d), jnp.float32),
    )(x, w, hp)


def _mid_body(p_ref, h_ref, hp_ref, sc_ref, w_ref, o_ref):
    dv = _dinv_block(hp_ref)
    u = (p_ref[0] + p_ref[1] + h_ref[...]) * dv
    z = jnp.maximum(u * sc_ref[0:1, :] + sc_ref[1:2, :], 0.0)
    o_ref[...] = jnp.dot(z, w_ref[...],
                         preferred_element_type=jnp.float32) * dv


def _tc_mid(p, h, hp, sc, w):
    n, d = h.shape
    br = _pick_br(n)
    return pl.pallas_call(
        _mid_body,
        grid=(n // br,),
        in_specs=[
            pl.BlockSpec((2, br, d), lambda i: (0, i, 0)),
            pl.BlockSpec((br, d), lambda i: (i, 0)),
            pl.BlockSpec((2, br, _L), lambda i: (0, i, 0)),
            pl.BlockSpec((2, d), lambda i: (0, 0)),
            pl.BlockSpec((d, d), lambda i: (0, 0)),
        ],
        out_specs=pl.BlockSpec((br, d), lambda i: (i, 0)),
        out_shape=jax.ShapeDtypeStruct((n, d), jnp.float32),
    )(p, h, hp, sc, w)


def _out_body(p_ref, h_ref, hp_ref, sc_ref, o_ref):
    dv = _dinv_block(hp_ref)
    u = (p_ref[0] + p_ref[1] + h_ref[...]) * dv
    o_ref[...] = jnp.maximum(u * sc_ref[0:1, :] + sc_ref[1:2, :], 0.0)


def _tc_out(p, h, hp, sc):
    n, d = h.shape
    br = _pick_br(n)
    return pl.pallas_call(
        _out_body,
        grid=(n // br,),
        in_specs=[
            pl.BlockSpec((2, br, d), lambda i: (0, i, 0)),
            pl.BlockSpec((br, d), lambda i: (i, 0)),
            pl.BlockSpec((2, br, _L), lambda i: (0, i, 0)),
            pl.BlockSpec((2, d), lambda i: (0, 0)),
        ],
        out_specs=pl.BlockSpec((br, d), lambda i: (i, 0)),
        out_shape=jax.ShapeDtypeStruct((n, d), jnp.float32),
    )(p, h, hp, sc)


def kernel(x, edge_index, W1, b1, g1, bt1, m1, v1, W2, b2, g2, bt2, m2, v2):
    n, d = x.shape
    e = edge_index.shape[1]
    assert e % _CHUNK == 0
    n_chunks = e // _CHUNK
    stripe = _NS * _CHUNK
    n_pad = ((n + stripe - 1) // stripe) * stripe

    s1 = g1 * lax.rsqrt(v1 + _EPS)
    sc1 = jnp.stack([s1, (b1 - m1) * s1 + bt1])
    s2 = g2 * lax.rsqrt(v2 + _EPS)
    sc2 = jnp.stack([s2, (b2 - m2) * s2 + bt2])

    hp = _sc_degree(edge_index, n_pad, n_chunks)
    h1 = _tc_matmul_scale(x, W1, hp)
    p1 = _sc_scatter(h1, edge_index, n_pad, n_chunks)
    h2 = _tc_mid(p1, h1, hp, sc1, W2)
    p2 = _sc_scatter(h2, edge_index, n_pad, n_chunks)
    return _tc_out(p2, h2, hp, sc2)


# speedup vs baseline: 15.0097x; 15.0097x over previous
import functools

import jax
import jax.numpy as jnp
from jax import lax
from jax.experimental import pallas as pl
from jax.experimental.pallas import tpu as pltpu
from jax.experimental.pallas import tpu_sc as plsc

_NC = 2
_NS = 16
_NW = _NC * _NS
_L = 16
_CHUNK = 128
_EPS = 1e-5


def _sc_mesh():
    return plsc.VectorSubcoreMesh(
        core_axis_name="c", subcore_axis_name="s",
        num_cores=_NC, num_subcores=_NS)


def _sc_degree(edge_index, n_pad, n_chunks, wd):
    rpt = n_pad // _NS

    @functools.partial(
        pl.kernel,
        out_type=jax.ShapeDtypeStruct((_NC, n_pad, wd), jnp.float32),
        mesh=_sc_mesh(),
        scratch_types=[
            pltpu.VMEM((_CHUNK,), jnp.int32),
            pltpu.VMEM((_CHUNK, wd), jnp.float32),
            pltpu.VMEM((_CHUNK, wd), jnp.float32),
            pltpu.VMEM_SHARED((n_pad, wd), jnp.float32),
        ],
    )
    def k(ei_hbm, out_hbm, idx_v, ones_v, zero_v, acc):
        cid = lax.axis_index("c")
        sid = lax.axis_index("s")
        w = sid * _NC + cid

        @pl.loop(0, _CHUNK)
        def _(r):
            for c0 in range(0, wd, _L):
                ones_v[r, pl.ds(c0, _L)] = jnp.ones((_L,), jnp.float32)
                zero_v[r, pl.ds(c0, _L)] = jnp.zeros((_L,), jnp.float32)

        @pl.loop(0, rpt, step=_CHUNK)
        def _(r0):
            pltpu.sync_copy(zero_v, acc.at[pl.ds(sid * rpt + r0, _CHUNK)])

        plsc.subcore_barrier()

        @pl.loop(w, n_chunks, step=_NW)
        def _(c):
            pltpu.sync_copy(ei_hbm.at[1, pl.ds(c * _CHUNK, _CHUNK)], idx_v)
            pltpu.sync_copy(ones_v, acc.at[idx_v], add=True)

        plsc.subcore_barrier()
        pltpu.sync_copy(acc.at[pl.ds(sid * rpt, rpt)],
                        out_hbm.at[cid, pl.ds(sid * rpt, rpt)])

    return k(edge_index)


def _sc_scatter(h, edge_index, n_pad, n_chunks):
    d = h.shape[1]
    rpt = n_pad // _NS

    @functools.partial(
        pl.kernel,
        out_type=jax.ShapeDtypeStruct((_NC, n_pad, d), jnp.float32),
        mesh=_sc_mesh(),
        scratch_types=[
            pltpu.VMEM((_CHUNK,), jnp.int32),
            pltpu.VMEM((_CHUNK,), jnp.int32),
            pltpu.VMEM((_CHUNK, d), jnp.float32),
            pltpu.VMEM_SHARED((n_pad, d), jnp.float32),
            pltpu.SemaphoreType.DMA,
        ],
    )
    def k(h_hbm, ei_hbm, out_hbm, src_v, dst_v, rows_v, acc, sem):
        cid = lax.axis_index("c")
        sid = lax.axis_index("s")
        w = sid * _NC + cid

        @pl.loop(0, _CHUNK)
        def _(r):
            for c0 in range(0, d, _L):
                rows_v[r, pl.ds(c0, _L)] = jnp.zeros((_L,), jnp.float32)

        @pl.loop(0, rpt, step=_CHUNK)
        def _(r0):
            pltpu.sync_copy(rows_v, acc.at[pl.ds(sid * rpt + r0, _CHUNK)])

        plsc.subcore_barrier()

        @pl.loop(w, n_chunks, step=_NW)
        def _(c):
            e0 = c * _CHUNK
            pltpu.sync_copy(ei_hbm.at[0, pl.ds(e0, _CHUNK)], src_v)
            pltpu.sync_copy(ei_hbm.at[1, pl.ds(e0, _CHUNK)], dst_v)
            pltpu.async_copy(h_hbm.at[src_v], rows_v, sem).wait()
            pltpu.sync_copy(rows_v, acc.at[dst_v], add=True)

        plsc.subcore_barrier()
        pltpu.sync_copy(acc.at[pl.ds(sid * rpt, rpt)],
                        out_hbm.at[cid, pl.ds(sid * rpt, rpt)])

    return k(h, edge_index)


def _pick_br(n):
    for br in (512, 400, 256, 200, 128, 80, 40, 16, 8):
        if n % br == 0:
            return br
    return n


def _dinv_block(hp_ref):
    deg = hp_ref[0, :, 0:1] + hp_ref[1, :, 0:1] + 1.0
    return lax.rsqrt(deg)


def _mm_body(x_ref, w_ref, hp_ref, o_ref):
    dv = _dinv_block(hp_ref)
    o_ref[...] = jnp.dot(x_ref[...], w_ref[...],
                         preferred_element_type=jnp.float32) * dv


def _tc_matmul_scale(x, w, hp):
    n, d = x.shape
    br = _pick_br(n)
    return pl.pallas_call(
        _mm_body,
        grid=(n // br,),
        in_specs=[
            pl.BlockSpec((br, d), lambda i: (i, 0)),
            pl.BlockSpec((d, d), lambda i: (0, 0)),
            pl.BlockSpec((2, br, d), lambda i: (0, i, 0)),
        ],
        out_specs=pl.BlockSpec((br, d), lambda i: (i, 0)),
        out_shape=jax.ShapeDtypeStruct((n, d), jnp.float32),
    )(x, w, hp)


def _mid_body(p_ref, h_ref, hp_ref, sc_ref, w_ref, o_ref):
    dv = _dinv_block(hp_ref)
    u = (p_ref[0] + p_ref[1] + h_ref[...]) * dv
    z = jnp.maximum(u * sc_ref[0:1, :] + sc_ref[1:2, :], 0.0)
    o_ref[...] = jnp.dot(z, w_ref[...],
                         preferred_element_type=jnp.float32) * dv


def _tc_mid(p, h, hp, sc, w):
    n, d = h.shape
    br = _pick_br(n)
    return pl.pallas_call(
        _mid_body,
        grid=(n // br,),
        in_specs=[
            pl.BlockSpec((2, br, d), lambda i: (0, i, 0)),
            pl.BlockSpec((br, d), lambda i: (i, 0)),
            pl.BlockSpec((2, br, d), lambda i: (0, i, 0)),
            pl.BlockSpec((2, d), lambda i: (0, 0)),
            pl.BlockSpec((d, d), lambda i: (0, 0)),
        ],
        out_specs=pl.BlockSpec((br, d), lambda i: (i, 0)),
        out_shape=jax.ShapeDtypeStruct((n, d), jnp.float32),
    )(p, h, hp, sc, w)


def _out_body(p_ref, h_ref, hp_ref, sc_ref, o_ref):
    dv = _dinv_block(hp_ref)
    u = (p_ref[0] + p_ref[1] + h_ref[...]) * dv
    o_ref[...] = jnp.maximum(u * sc_ref[0:1, :] + sc_ref[1:2, :], 0.0)


def _tc_out(p, h, hp, sc):
    n, d = h.shape
    br = _pick_br(n)
    return pl.pallas_call(
        _out_body,
        grid=(n // br,),
        in_specs=[
            pl.BlockSpec((2, br, d), lambda i: (0, i, 0)),
            pl.BlockSpec((br, d), lambda i: (i, 0)),
            pl.BlockSpec((2, br, d), lambda i: (0, i, 0)),
            pl.BlockSpec((2, d), lambda i: (0, 0)),
        ],
        out_specs=pl.BlockSpec((br, d), lambda i: (i, 0)),
        out_shape=jax.ShapeDtypeStruct((n, d), jnp.float32),
    )(p, h, hp, sc)


def kernel(x, edge_index, W1, b1, g1, bt1, m1, v1, W2, b2, g2, bt2, m2, v2):
    n, d = x.shape
    e = edge_index.shape[1]
    assert e % _CHUNK == 0
    n_chunks = e // _CHUNK
    stripe = _NS * _CHUNK
    n_pad = ((n + stripe - 1) // stripe) * stripe

    s1 = g1 * lax.rsqrt(v1 + _EPS)
    sc1 = jnp.stack([s1, (b1 - m1) * s1 + bt1])
    s2 = g2 * lax.rsqrt(v2 + _EPS)
    sc2 = jnp.stack([s2, (b2 - m2) * s2 + bt2])

    hp = _sc_degree(edge_index, n_pad, n_chunks, d)
    h1 = _tc_matmul_scale(x, W1, hp)
    p1 = _sc_scatter(h1, edge_index, n_pad, n_chunks)
    h2 = _tc_mid(p1, h1, hp, sc1, W2)
    p2 = _sc_scatter(h2, edge_index, n_pad, n_chunks)
    return _tc_out(p2, h2, hp, sc2)


# speedup vs baseline: 23.1038x; 1.5393x over previous
import functools

import jax
import jax.numpy as jnp
from jax import lax
from jax.experimental import pallas as pl
from jax.experimental.pallas import tpu as pltpu
from jax.experimental.pallas import tpu_sc as plsc

_NC = 2
_NS = 16
_NW = _NC * _NS
_L = 16
_CHUNK = 128
_EPS = 1e-5


def _sc_mesh():
    return plsc.VectorSubcoreMesh(
        core_axis_name="c", subcore_axis_name="s",
        num_cores=_NC, num_subcores=_NS)


def _sc_degree(edge_index, n_pad, n_chunks, wd):
    rpt = n_pad // _NS

    @functools.partial(
        pl.kernel,
        out_type=jax.ShapeDtypeStruct((_NC, n_pad, wd), jnp.float32),
        mesh=_sc_mesh(),
        scratch_types=[
            pltpu.VMEM((_CHUNK,), jnp.int32),
            pltpu.VMEM((_CHUNK, wd), jnp.float32),
            pltpu.VMEM((_CHUNK, wd), jnp.float32),
            pltpu.VMEM_SHARED((n_pad, wd), jnp.float32),
        ],
    )
    def k(ei_hbm, out_hbm, idx_v, ones_v, zero_v, acc):
        cid = lax.axis_index("c")
        sid = lax.axis_index("s")
        w = sid * _NC + cid

        @pl.loop(0, _CHUNK)
        def _(r):
            for c0 in range(0, wd, _L):
                ones_v[r, pl.ds(c0, _L)] = jnp.ones((_L,), jnp.float32)
                zero_v[r, pl.ds(c0, _L)] = jnp.zeros((_L,), jnp.float32)

        @pl.loop(0, rpt, step=_CHUNK)
        def _(r0):
            pltpu.sync_copy(zero_v, acc.at[pl.ds(sid * rpt + r0, _CHUNK)])

        plsc.subcore_barrier()

        @pl.loop(w, n_chunks, step=_NW)
        def _(c):
            pltpu.sync_copy(ei_hbm.at[1, pl.ds(c * _CHUNK, _CHUNK)], idx_v)
            pltpu.sync_copy(ones_v, acc.at[idx_v], add=True)

        plsc.subcore_barrier()
        pltpu.sync_copy(acc.at[pl.ds(sid * rpt, rpt)],
                        out_hbm.at[cid, pl.ds(sid * rpt, rpt)])

    return k(edge_index)


def _sc_scatter(h, edge_index, n_pad, n_chunks):
    d = h.shape[1]
    rpt = n_pad // _NS

    jmax = (n_chunks // _NW) // 2 + 2

    @functools.partial(
        pl.kernel,
        out_type=jax.ShapeDtypeStruct((_NC, n_pad, d), jnp.float32),
        mesh=_sc_mesh(),
        scratch_types=[
            pltpu.VMEM((2, _CHUNK), jnp.int32),
            pltpu.VMEM((2, _CHUNK), jnp.int32),
            pltpu.VMEM((_CHUNK,), jnp.int32),
            pltpu.VMEM((_CHUNK,), jnp.int32),
            pltpu.VMEM((_CHUNK, d), jnp.float32),
            pltpu.VMEM((_CHUNK, d), jnp.float32),
            pltpu.VMEM_SHARED((n_pad, d), jnp.float32),
            pltpu.SemaphoreType.DMA,
            pltpu.SemaphoreType.DMA,
            pltpu.SemaphoreType.DMA,
            pltpu.SemaphoreType.DMA,
            pltpu.SemaphoreType.DMA,
        ],
    )
    def k(h_hbm, ei_hbm, out_hbm, idx0, idx1, dst0, dst1, rows0, rows1,
          acc, sem_i0, sem_i1, sem_g, sem_s0, sem_s1):
        cid = lax.axis_index("c")
        sid = lax.axis_index("s")
        w = sid * _NC + cid
        k_t = (n_chunks - 1 - w) // _NW + 1
        last_c = n_chunks - 1

        def idx_start(c, idx_v, sem):
            pltpu.async_copy(ei_hbm.at[:, pl.ds(c * _CHUNK, _CHUNK)],
                             idx_v, sem)

        def idx_wait(idx_v, sem):
            pltpu.make_async_copy(ei_hbm.at[:, pl.ds(0, _CHUNK)],
                                  idx_v, sem).wait()

        idx_start(w, idx0, sem_i0)

        @pl.loop(0, _CHUNK)
        def _(r):
            for c0 in range(0, d, _L):
                rows0[r, pl.ds(c0, _L)] = jnp.zeros((_L,), jnp.float32)

        @pl.loop(0, rpt, step=_CHUNK)
        def _(r0):
            pltpu.sync_copy(rows0, acc.at[pl.ds(sid * rpt + r0, _CHUNK)])

        plsc.subcore_barrier()

        def item(k_idx, idx_a, sem_ia, dst_a, rows_a, sem_sa, idx_b, sem_ib,
                 first):
            @pl.when(k_idx < k_t)
            def _():
                c_next = jnp.minimum(w + (k_idx + 1) * _NW, last_c)
                idx_start(c_next, idx_b, sem_ib)
                if not first:
                    pltpu.make_async_copy(rows_a, acc.at[dst_a],
                                          sem_sa).wait()
                idx_wait(idx_a, sem_ia)
                pltpu.async_copy(h_hbm.at[idx_a.at[0]], rows_a, sem_g).wait()
                for c0 in range(0, _CHUNK, _L):
                    dst_a[pl.ds(c0, _L)] = idx_a[1, pl.ds(c0, _L)]
                pltpu.async_copy(rows_a, acc.at[dst_a], sem_sa, add=True)

        item(0, idx0, sem_i0, dst0, rows0, sem_s0, idx1, sem_i1, True)
        item(1, idx1, sem_i1, dst1, rows1, sem_s1, idx0, sem_i0, True)

        @pl.loop(1, jmax)
        def _(j):
            item(2 * j, idx0, sem_i0, dst0, rows0, sem_s0,
                 idx1, sem_i1, False)
            item(2 * j + 1, idx1, sem_i1, dst1, rows1, sem_s1,
                 idx0, sem_i0, False)

        pltpu.make_async_copy(rows0, acc.at[dst0], sem_s0).wait()
        pltpu.make_async_copy(rows1, acc.at[dst1], sem_s1).wait()

        @pl.when(k_t % 2 == 0)
        def _():
            idx_wait(idx0, sem_i0)

        @pl.when(k_t % 2 == 1)
        def _():
            idx_wait(idx1, sem_i1)

        plsc.subcore_barrier()
        pltpu.sync_copy(acc.at[pl.ds(sid * rpt, rpt)],
                        out_hbm.at[cid, pl.ds(sid * rpt, rpt)])

    return k(h, edge_index)


def _pick_br(n):
    for br in (512, 400, 256, 200, 128, 80, 40, 16, 8):
        if n % br == 0:
            return br
    return n


def _dinv_block(hp_ref):
    deg = hp_ref[0, :, 0:1] + hp_ref[1, :, 0:1] + 1.0
    return lax.rsqrt(deg)


def _mm_body(x_ref, w_ref, hp_ref, o_ref):
    dv = _dinv_block(hp_ref)
    o_ref[...] = jnp.dot(x_ref[...], w_ref[...],
                         preferred_element_type=jnp.float32) * dv


def _tc_matmul_scale(x, w, hp):
    n, d = x.shape
    br = _pick_br(n)
    return pl.pallas_call(
        _mm_body,
        grid=(n // br,),
        in_specs=[
            pl.BlockSpec((br, d), lambda i: (i, 0)),
            pl.BlockSpec((d, d), lambda i: (0, 0)),
            pl.BlockSpec((2, br, d), lambda i: (0, i, 0)),
        ],
        out_specs=pl.BlockSpec((br, d), lambda i: (i, 0)),
        out_shape=jax.ShapeDtypeStruct((n, d), jnp.float32),
    )(x, w, hp)


def _mid_body(p_ref, h_ref, hp_ref, sc_ref, w_ref, o_ref):
    dv = _dinv_block(hp_ref)
    u = (p_ref[0] + p_ref[1] + h_ref[...]) * dv
    z = jnp.maximum(u * sc_ref[0:1, :] + sc_ref[1:2, :], 0.0)
    o_ref[...] = jnp.dot(z, w_ref[...],
                         preferred_element_type=jnp.float32) * dv


def _tc_mid(p, h, hp, sc, w):
    n, d = h.shape
    br = _pick_br(n)
    return pl.pallas_call(
        _mid_body,
        grid=(n // br,),
        in_specs=[
            pl.BlockSpec((2, br, d), lambda i: (0, i, 0)),
            pl.BlockSpec((br, d), lambda i: (i, 0)),
            pl.BlockSpec((2, br, d), lambda i: (0, i, 0)),
            pl.BlockSpec((2, d), lambda i: (0, 0)),
            pl.BlockSpec((d, d), lambda i: (0, 0)),
        ],
        out_specs=pl.BlockSpec((br, d), lambda i: (i, 0)),
        out_shape=jax.ShapeDtypeStruct((n, d), jnp.float32),
    )(p, h, hp, sc, w)


def _out_body(p_ref, h_ref, hp_ref, sc_ref, o_ref):
    dv = _dinv_block(hp_ref)
    u = (p_ref[0] + p_ref[1] + h_ref[...]) * dv
    o_ref[...] = jnp.maximum(u * sc_ref[0:1, :] + sc_ref[1:2, :], 0.0)


def _tc_out(p, h, hp, sc):
    n, d = h.shape
    br = _pick_br(n)
    return pl.pallas_call(
        _out_body,
        grid=(n // br,),
        in_specs=[
            pl.BlockSpec((2, br, d), lambda i: (0, i, 0)),
            pl.BlockSpec((br, d), lambda i: (i, 0)),
            pl.BlockSpec((2, br, d), lambda i: (0, i, 0)),
            pl.BlockSpec((2, d), lambda i: (0, 0)),
        ],
        out_specs=pl.BlockSpec((br, d), lambda i: (i, 0)),
        out_shape=jax.ShapeDtypeStruct((n, d), jnp.float32),
    )(p, h, hp, sc)


def kernel(x, edge_index, W1, b1, g1, bt1, m1, v1, W2, b2, g2, bt2, m2, v2):
    n, d = x.shape
    e = edge_index.shape[1]
    assert e % _CHUNK == 0
    n_chunks = e // _CHUNK
    stripe = _NS * _CHUNK
    n_pad = ((n + stripe - 1) // stripe) * stripe

    s1 = g1 * lax.rsqrt(v1 + _EPS)
    sc1 = jnp.stack([s1, (b1 - m1) * s1 + bt1])
    s2 = g2 * lax.rsqrt(v2 + _EPS)
    sc2 = jnp.stack([s2, (b2 - m2) * s2 + bt2])

    hp = _sc_degree(edge_index, n_pad, n_chunks, d)
    h1 = _tc_matmul_scale(x, W1, hp)
    p1 = _sc_scatter(h1, edge_index, n_pad, n_chunks)
    h2 = _tc_mid(p1, h1, hp, sc1, W2)
    p2 = _sc_scatter(h2, edge_index, n_pad, n_chunks)
    return _tc_out(p2, h2, hp, sc2)


# speedup vs baseline: 27.1875x; 1.1768x over previous
import dataclasses
import functools

import jax
import jax.numpy as jnp
from jax import lax
from jax.experimental import pallas as pl
from jax.experimental.pallas import tpu as pltpu
from jax.experimental.pallas import tpu_sc as plsc

_NC = 2
_NS = 16
_NW = _NC * _NS
_L = 16
_CHUNK = 128
_EPS = 1e-5


def _sc_mesh():
    return plsc.VectorSubcoreMesh(
        core_axis_name="c", subcore_axis_name="s",
        num_cores=_NC, num_subcores=_NS)


def _sc_degree(edge_index, n_pad, n_chunks):
    rpt = n_pad // _NS
    jmax = (n_chunks // _NW) // 2 + 2

    @functools.partial(
        pl.kernel,
        out_type=jax.ShapeDtypeStruct((_NC, n_pad), jnp.float32),
        mesh=_sc_mesh(),
        compiler_params=dataclasses.replace(pltpu.CompilerParams(),
                                            needs_layout_passes=False),
        scratch_types=[
            pltpu.VMEM((_CHUNK,), jnp.int32),
            pltpu.VMEM((_CHUNK,), jnp.int32),
            pltpu.VMEM((n_pad,), jnp.float32),
            pltpu.VMEM((_NS, rpt), jnp.float32),
            pltpu.VMEM_SHARED((_NS, n_pad), jnp.float32),
            pltpu.SemaphoreType.DMA,
            pltpu.SemaphoreType.DMA,
        ],
    )
    def k(ei_hbm, out_hbm, idx0, idx1, acc_t, red_v, shared, sem_i0, sem_i1):
        cid = lax.axis_index("c")
        sid = lax.axis_index("s")
        w = sid * _NC + cid
        k_t = (n_chunks - 1 - w) // _NW + 1
        last_c = n_chunks - 1
        ones = jnp.ones((_L,), jnp.float32)

        def idx_start(c, idx_v, sem):
            pltpu.async_copy(ei_hbm.at[1, pl.ds(c * _CHUNK, _CHUNK)],
                             idx_v, sem)

        def idx_wait(idx_v, sem):
            pltpu.make_async_copy(ei_hbm.at[1, pl.ds(0, _CHUNK)],
                                  idx_v, sem).wait()

        idx_start(w, idx0, sem_i0)

        @pl.loop(0, n_pad, step=_L)
        def _(r0):
            acc_t[pl.ds(r0, _L)] = jnp.zeros((_L,), jnp.float32)

        def item(k_idx, idx_a, sem_a, idx_b, sem_b):
            @pl.when(k_idx < k_t)
            def _():
                c_next = jnp.minimum(w + (k_idx + 1) * _NW, last_c)
                idx_start(c_next, idx_b, sem_b)
                idx_wait(idx_a, sem_a)
                for c0 in range(0, _CHUNK, _L):
                    plsc.addupdate_scatter(
                        acc_t, [idx_a[pl.ds(c0, _L)]], ones)

        item(0, idx0, sem_i0, idx1, sem_i1)
        item(1, idx1, sem_i1, idx0, sem_i0)

        @pl.loop(1, jmax)
        def _(j):
            item(2 * j, idx0, sem_i0, idx1, sem_i1)
            item(2 * j + 1, idx1, sem_i1, idx0, sem_i0)

        @pl.when(k_t % 2 == 0)
        def _():
            idx_wait(idx0, sem_i0)

        @pl.when(k_t % 2 == 1)
        def _():
            idx_wait(idx1, sem_i1)

        pltpu.sync_copy(acc_t, shared.at[sid])
        plsc.subcore_barrier()
        col0 = sid * rpt
        pltpu.sync_copy(shared.at[:, pl.ds(col0, rpt)], red_v)
        for r in range(1, _NS):
            @pl.loop(0, rpt, step=_L)
            def _(c0, _r=r):
                red_v[0, pl.ds(c0, _L)] = (red_v[0, pl.ds(c0, _L)]
                                           + red_v[_r, pl.ds(c0, _L)])
        pltpu.sync_copy(red_v.at[0], out_hbm.at[cid, pl.ds(col0, rpt)])

    return k(edge_index)


def _sc_scatter(h, edge_index, n_pad, n_chunks):
    d = h.shape[1]
    rpt = n_pad // _NS

    jmax = (n_chunks // _NW) // 2 + 2

    @functools.partial(
        pl.kernel,
        out_type=jax.ShapeDtypeStruct((_NC, n_pad, d), jnp.float32),
        mesh=_sc_mesh(),
        scratch_types=[
            pltpu.VMEM((2, _CHUNK), jnp.int32),
            pltpu.VMEM((2, _CHUNK), jnp.int32),
            pltpu.VMEM((_CHUNK,), jnp.int32),
            pltpu.VMEM((_CHUNK,), jnp.int32),
            pltpu.VMEM((_CHUNK, d), jnp.float32),
            pltpu.VMEM((_CHUNK, d), jnp.float32),
            pltpu.VMEM_SHARED((n_pad, d), jnp.float32),
            pltpu.SemaphoreType.DMA,
            pltpu.SemaphoreType.DMA,
            pltpu.SemaphoreType.DMA,
            pltpu.SemaphoreType.DMA,
            pltpu.SemaphoreType.DMA,
        ],
    )
    def k(h_hbm, ei_hbm, out_hbm, idx0, idx1, dst0, dst1, rows0, rows1,
          acc, sem_i0, sem_i1, sem_g, sem_s0, sem_s1):
        cid = lax.axis_index("c")
        sid = lax.axis_index("s")
        w = sid * _NC + cid
        k_t = (n_chunks - 1 - w) // _NW + 1
        last_c = n_chunks - 1

        def idx_start(c, idx_v, sem):
            pltpu.async_copy(ei_hbm.at[:, pl.ds(c * _CHUNK, _CHUNK)],
                             idx_v, sem)

        def idx_wait(idx_v, sem):
            pltpu.make_async_copy(ei_hbm.at[:, pl.ds(0, _CHUNK)],
                                  idx_v, sem).wait()

        idx_start(w, idx0, sem_i0)

        @pl.loop(0, _CHUNK)
        def _(r):
            for c0 in range(0, d, _L):
                rows0[r, pl.ds(c0, _L)] = jnp.zeros((_L,), jnp.float32)

        @pl.loop(0, rpt, step=_CHUNK)
        def _(r0):
            pltpu.sync_copy(rows0, acc.at[pl.ds(sid * rpt + r0, _CHUNK)])

        plsc.subcore_barrier()

        def item(k_idx, idx_a, sem_ia, dst_a, rows_a, sem_sa, idx_b, sem_ib,
                 first):
            @pl.when(k_idx < k_t)
            def _():
                c_next = jnp.minimum(w + (k_idx + 1) * _NW, last_c)
                idx_start(c_next, idx_b, sem_ib)
                if not first:
                    pltpu.make_async_copy(rows_a, acc.at[dst_a],
                                          sem_sa).wait()
                idx_wait(idx_a, sem_ia)
                pltpu.async_copy(h_hbm.at[idx_a.at[0]], rows_a, sem_g).wait()
                for c0 in range(0, _CHUNK, _L):
                    dst_a[pl.ds(c0, _L)] = idx_a[1, pl.ds(c0, _L)]
                pltpu.async_copy(rows_a, acc.at[dst_a], sem_sa, add=True)

        item(0, idx0, sem_i0, dst0, rows0, sem_s0, idx1, sem_i1, True)
        item(1, idx1, sem_i1, dst1, rows1, sem_s1, idx0, sem_i0, True)

        @pl.loop(1, jmax)
        def _(j):
            item(2 * j, idx0, sem_i0, dst0, rows0, sem_s0,
                 idx1, sem_i1, False)
            item(2 * j + 1, idx1, sem_i1, dst1, rows1, sem_s1,
                 idx0, sem_i0, False)

        pltpu.make_async_copy(rows0, acc.at[dst0], sem_s0).wait()
        pltpu.make_async_copy(rows1, acc.at[dst1], sem_s1).wait()

        @pl.when(k_t % 2 == 0)
        def _():
            idx_wait(idx0, sem_i0)

        @pl.when(k_t % 2 == 1)
        def _():
            idx_wait(idx1, sem_i1)

        plsc.subcore_barrier()
        pltpu.sync_copy(acc.at[pl.ds(sid * rpt, rpt)],
                        out_hbm.at[cid, pl.ds(sid * rpt, rpt)])

    return k(h, edge_index)


def _pick_br(n):
    for br in (512, 400, 256, 200, 128, 80, 40, 16, 8):
        if n % br == 0:
            return br
    return n


def _dinv_block(hp_ref):
    deg = hp_ref[0] + hp_ref[1] + 1.0
    return lax.rsqrt(deg)


def _mm_body(x_ref, w_ref, hp_ref, o_ref):
    dv = _dinv_block(hp_ref)
    o_ref[...] = jnp.dot(x_ref[...], w_ref[...],
                         preferred_element_type=jnp.float32) * dv


def _tc_matmul_scale(x, w, hp):
    n, d = x.shape
    br = _pick_br(n)
    return pl.pallas_call(
        _mm_body,
        grid=(n // br,),
        in_specs=[
            pl.BlockSpec((br, d), lambda i: (i, 0)),
            pl.BlockSpec((d, d), lambda i: (0, 0)),
            pl.BlockSpec((2, br, 1), lambda i: (0, i, 0)),
        ],
        out_specs=pl.BlockSpec((br, d), lambda i: (i, 0)),
        out_shape=jax.ShapeDtypeStruct((n, d), jnp.float32),
    )(x, w, hp)


def _mid_body(p_ref, h_ref, hp_ref, sc_ref, w_ref, o_ref):
    dv = _dinv_block(hp_ref)
    u = (p_ref[0] + p_ref[1] + h_ref[...]) * dv
    z = jnp.maximum(u * sc_ref[0:1, :] + sc_ref[1:2, :], 0.0)
    o_ref[...] = jnp.dot(z, w_ref[...],
                         preferred_element_type=jnp.float32) * dv


def _tc_mid(p, h, hp, sc, w):
    n, d = h.shape
    br = _pick_br(n)
    return pl.pallas_call(
        _mid_body,
        grid=(n // br,),
        in_specs=[
            pl.BlockSpec((2, br, d), lambda i: (0, i, 0)),
            pl.BlockSpec((br, d), lambda i: (i, 0)),
            pl.BlockSpec((2, br, 1), lambda i: (0, i, 0)),
            pl.BlockSpec((2, d), lambda i: (0, 0)),
            pl.BlockSpec((d, d), lambda i: (0, 0)),
        ],
        out_specs=pl.BlockSpec((br, d), lambda i: (i, 0)),
        out_shape=jax.ShapeDtypeStruct((n, d), jnp.float32),
    )(p, h, hp, sc, w)


def _out_body(p_ref, h_ref, hp_ref, sc_ref, o_ref):
    dv = _dinv_block(hp_ref)
    u = (p_ref[0] + p_ref[1] + h_ref[...]) * dv
    o_ref[...] = jnp.maximum(u * sc_ref[0:1, :] + sc_ref[1:2, :], 0.0)


def _tc_out(p, h, hp, sc):
    n, d = h.shape
    br = _pick_br(n)
    return pl.pallas_call(
        _out_body,
        grid=(n // br,),
        in_specs=[
            pl.BlockSpec((2, br, d), lambda i: (0, i, 0)),
            pl.BlockSpec((br, d), lambda i: (i, 0)),
            pl.BlockSpec((2, br, 1), lambda i: (0, i, 0)),
            pl.BlockSpec((2, d), lambda i: (0, 0)),
        ],
        out_specs=pl.BlockSpec((br, d), lambda i: (i, 0)),
        out_shape=jax.ShapeDtypeStruct((n, d), jnp.float32),
    )(p, h, hp, sc)


def kernel(x, edge_index, W1, b1, g1, bt1, m1, v1, W2, b2, g2, bt2, m2, v2):
    n, d = x.shape
    e = edge_index.shape[1]
    assert e % _CHUNK == 0
    n_chunks = e // _CHUNK
    stripe = _NS * _CHUNK
    n_pad = ((n + stripe - 1) // stripe) * stripe

    s1 = g1 * lax.rsqrt(v1 + _EPS)
    sc1 = jnp.stack([s1, (b1 - m1) * s1 + bt1])
    s2 = g2 * lax.rsqrt(v2 + _EPS)
    sc2 = jnp.stack([s2, (b2 - m2) * s2 + bt2])

    hp = _sc_degree(edge_index, n_pad, n_chunks)
    hp = hp.reshape(_NC, n_pad, 1)
    h1 = _tc_matmul_scale(x, W1, hp)
    p1 = _sc_scatter(h1, edge_index, n_pad, n_chunks)
    h2 = _tc_mid(p1, h1, hp, sc1, W2)
    p2 = _sc_scatter(h2, edge_index, n_pad, n_chunks)
    return _tc_out(p2, h2, hp, sc2)


# speedup vs baseline: 31.1497x; 1.1457x over previous
import dataclasses
import functools

import jax
import jax.numpy as jnp
from jax import lax
from jax.experimental import pallas as pl
from jax.experimental.pallas import tpu as pltpu
from jax.experimental.pallas import tpu_sc as plsc

_NC = 2
_NS = 16
_NW = _NC * _NS
_L = 16
_CHUNK = 128
_EPS = 1e-5


def _sc_mesh():
    return plsc.VectorSubcoreMesh(
        core_axis_name="c", subcore_axis_name="s",
        num_cores=_NC, num_subcores=_NS)


def _sc_degree(edge_index, n_pad, n_chunks):
    rpt = n_pad // _NS
    jmax = (n_chunks // _NW) // 2 + 2

    @functools.partial(
        pl.kernel,
        out_type=jax.ShapeDtypeStruct((_NC, n_pad), jnp.float32),
        mesh=_sc_mesh(),
        compiler_params=dataclasses.replace(pltpu.CompilerParams(),
                                            needs_layout_passes=False),
        scratch_types=[
            pltpu.VMEM((_CHUNK,), jnp.int32),
            pltpu.VMEM((_CHUNK,), jnp.int32),
            pltpu.VMEM((n_pad,), jnp.float32),
            pltpu.VMEM((_NS, rpt), jnp.float32),
            pltpu.VMEM_SHARED((_NS, n_pad), jnp.float32),
            pltpu.SemaphoreType.DMA,
            pltpu.SemaphoreType.DMA,
        ],
    )
    def k(ei_hbm, out_hbm, idx0, idx1, acc_t, red_v, shared, sem_i0, sem_i1):
        cid = lax.axis_index("c")
        sid = lax.axis_index("s")
        w = sid * _NC + cid
        k_t = (n_chunks - 1 - w) // _NW + 1
        last_c = n_chunks - 1
        ones = jnp.ones((_L,), jnp.float32)

        def idx_start(c, idx_v, sem):
            pltpu.async_copy(ei_hbm.at[1, pl.ds(c * _CHUNK, _CHUNK)],
                             idx_v, sem)

        def idx_wait(idx_v, sem):
            pltpu.make_async_copy(ei_hbm.at[1, pl.ds(0, _CHUNK)],
                                  idx_v, sem).wait()

        idx_start(w, idx0, sem_i0)

        @pl.loop(0, n_pad, step=_L)
        def _(r0):
            acc_t[pl.ds(r0, _L)] = jnp.zeros((_L,), jnp.float32)

        def item(k_idx, idx_a, sem_a, idx_b, sem_b):
            @pl.when(k_idx < k_t)
            def _():
                c_next = jnp.minimum(w + (k_idx + 1) * _NW, last_c)
                idx_start(c_next, idx_b, sem_b)
                idx_wait(idx_a, sem_a)
                for c0 in range(0, _CHUNK, _L):
                    plsc.addupdate_scatter(
                        acc_t, [idx_a[pl.ds(c0, _L)]], ones)

        item(0, idx0, sem_i0, idx1, sem_i1)
        item(1, idx1, sem_i1, idx0, sem_i0)

        @pl.loop(1, jmax)
        def _(j):
            item(2 * j, idx0, sem_i0, idx1, sem_i1)
            item(2 * j + 1, idx1, sem_i1, idx0, sem_i0)

        @pl.when(k_t % 2 == 0)
        def _():
            idx_wait(idx0, sem_i0)

        @pl.when(k_t % 2 == 1)
        def _():
            idx_wait(idx1, sem_i1)

        pltpu.sync_copy(acc_t, shared.at[sid])
        plsc.subcore_barrier()
        col0 = sid * rpt
        pltpu.sync_copy(shared.at[:, pl.ds(col0, rpt)], red_v)
        for r in range(1, _NS):
            @pl.loop(0, rpt, step=_L)
            def _(c0, _r=r):
                red_v[0, pl.ds(c0, _L)] = (red_v[0, pl.ds(c0, _L)]
                                           + red_v[_r, pl.ds(c0, _L)])
        pltpu.sync_copy(red_v.at[0], out_hbm.at[cid, pl.ds(col0, rpt)])

    return k(edge_index)


def _sc_scatter(h, edge_index, n_pad, n_chunks):
    d = h.shape[1]
    rpt = n_pad // _NS

    jmax = (n_chunks // _NW) // 2 + 2

    @functools.partial(
        pl.kernel,
        out_type=jax.ShapeDtypeStruct((_NC, n_pad, d), jnp.float32),
        mesh=_sc_mesh(),
        scratch_types=[
            pltpu.VMEM((2, _CHUNK), jnp.int32),
            pltpu.VMEM((2, _CHUNK), jnp.int32),
            pltpu.VMEM((_CHUNK,), jnp.int32),
            pltpu.VMEM((_CHUNK,), jnp.int32),
            pltpu.VMEM((_CHUNK,), jnp.int32),
            pltpu.VMEM((_CHUNK,), jnp.int32),
            pltpu.VMEM((_CHUNK, d), jnp.float32),
            pltpu.VMEM((_CHUNK, d), jnp.float32),
            pltpu.VMEM_SHARED((n_pad, d), jnp.float32),
            pltpu.SemaphoreType.DMA,
            pltpu.SemaphoreType.DMA,
            pltpu.SemaphoreType.DMA,
            pltpu.SemaphoreType.DMA,
            pltpu.SemaphoreType.DMA,
            pltpu.SemaphoreType.DMA,
        ],
    )
    def k(h_hbm, ei_hbm, out_hbm, idx0, idx1, ssh0, ssh1, dsh0, dsh1,
          rows0, rows1, acc, sem_i0, sem_i1, sem_g0, sem_g1,
          sem_s0, sem_s1):
        cid = lax.axis_index("c")
        sid = lax.axis_index("s")
        w = sid * _NC + cid
        k_t = (n_chunks - 1 - w) // _NW + 1
        last_c = n_chunks - 1

        def idx_start(c, idx_v, sem):
            pltpu.async_copy(ei_hbm.at[:, pl.ds(c * _CHUNK, _CHUNK)],
                             idx_v, sem)

        def idx_wait(idx_v, sem):
            pltpu.make_async_copy(ei_hbm.at[:, pl.ds(0, _CHUNK)],
                                  idx_v, sem).wait()

        idx_start(w, idx0, sem_i0)

        @pl.loop(0, _CHUNK)
        def _(r):
            for c0 in range(0, d, _L):
                rows0[r, pl.ds(c0, _L)] = jnp.zeros((_L,), jnp.float32)

        @pl.loop(0, rpt, step=_CHUNK)
        def _(r0):
            pltpu.sync_copy(rows0, acc.at[pl.ds(sid * rpt + r0, _CHUNK)])

        plsc.subcore_barrier()

        def item(k_idx, idx_a, sem_ia, ssh_a, dsh_a, rows_a, sem_ga, sem_sa,
                 dsh_b, rows_b, sem_gb, sem_sb, idx_b, sem_ib, first,
                 do_prev):
            @pl.when(k_idx < k_t)
            def _():
                c_next = jnp.minimum(w + (k_idx + 1) * _NW, last_c)
                idx_start(c_next, idx_b, sem_ib)
                idx_wait(idx_a, sem_ia)
                if not first:
                    pltpu.make_async_copy(rows_a, acc.at[dsh_a],
                                          sem_sa).wait()
                for c0 in range(0, _CHUNK, _L):
                    ssh_a[pl.ds(c0, _L)] = idx_a[0, pl.ds(c0, _L)]
                    dsh_a[pl.ds(c0, _L)] = idx_a[1, pl.ds(c0, _L)]
                pltpu.async_copy(h_hbm.at[ssh_a], rows_a, sem_ga)
                if do_prev:
                    pltpu.make_async_copy(h_hbm.at[ssh_a], rows_b,
                                          sem_gb).wait()
                    pltpu.async_copy(rows_b, acc.at[dsh_b], sem_sb,
                                     add=True)

        item(0, idx0, sem_i0, ssh0, dsh0, rows0, sem_g0, sem_s0,
             dsh1, rows1, sem_g1, sem_s1, idx1, sem_i1, True, False)
        item(1, idx1, sem_i1, ssh1, dsh1, rows1, sem_g1, sem_s1,
             dsh0, rows0, sem_g0, sem_s0, idx0, sem_i0, True, True)

        @pl.loop(1, jmax)
        def _(j):
            item(2 * j, idx0, sem_i0, ssh0, dsh0, rows0, sem_g0, sem_s0,
                 dsh1, rows1, sem_g1, sem_s1, idx1, sem_i1, False, True)
            item(2 * j + 1, idx1, sem_i1, ssh1, dsh1, rows1, sem_g1, sem_s1,
                 dsh0, rows0, sem_g0, sem_s0, idx0, sem_i0, False, True)

        @pl.when(k_t % 2 == 1)
        def _():
            pltpu.make_async_copy(h_hbm.at[ssh0], rows0, sem_g0).wait()
            pltpu.async_copy(rows0, acc.at[dsh0], sem_s0, add=True)
            idx_wait(idx1, sem_i1)

        @pl.when(k_t % 2 == 0)
        def _():
            pltpu.make_async_copy(h_hbm.at[ssh1], rows1, sem_g1).wait()
            pltpu.async_copy(rows1, acc.at[dsh1], sem_s1, add=True)
            idx_wait(idx0, sem_i0)

        pltpu.make_async_copy(rows0, acc.at[dsh0], sem_s0).wait()
        pltpu.make_async_copy(rows1, acc.at[dsh1], sem_s1).wait()

        plsc.subcore_barrier()
        pltpu.sync_copy(acc.at[pl.ds(sid * rpt, rpt)],
                        out_hbm.at[cid, pl.ds(sid * rpt, rpt)])

    return k(h, edge_index)


def _pick_br(n):
    for br in (512, 400, 256, 200, 128, 80, 40, 16, 8):
        if n % br == 0:
            return br
    return n


def _dinv_block(hp_ref):
    deg = hp_ref[0] + hp_ref[1] + 1.0
    return lax.rsqrt(deg)


def _mm_body(x_ref, w_ref, hp_ref, o_ref):
    dv = _dinv_block(hp_ref)
    o_ref[...] = jnp.dot(x_ref[...], w_ref[...],
                         preferred_element_type=jnp.float32) * dv


def _tc_matmul_scale(x, w, hp):
    n, d = x.shape
    br = _pick_br(n)
    return pl.pallas_call(
        _mm_body,
        grid=(n // br,),
        in_specs=[
            pl.BlockSpec((br, d), lambda i: (i, 0)),
            pl.BlockSpec((d, d), lambda i: (0, 0)),
            pl.BlockSpec((2, br, 1), lambda i: (0, i, 0)),
        ],
        out_specs=pl.BlockSpec((br, d), lambda i: (i, 0)),
        out_shape=jax.ShapeDtypeStruct((n, d), jnp.float32),
    )(x, w, hp)


def _mid_body(p_ref, h_ref, hp_ref, sc_ref, w_ref, o_ref):
    dv = _dinv_block(hp_ref)
    u = (p_ref[0] + p_ref[1] + h_ref[...]) * dv
    z = jnp.maximum(u * sc_ref[0:1, :] + sc_ref[1:2, :], 0.0)
    o_ref[...] = jnp.dot(z, w_ref[...],
                         preferred_element_type=jnp.float32) * dv


def _tc_mid(p, h, hp, sc, w):
    n, d = h.shape
    br = _pick_br(n)
    return pl.pallas_call(
        _mid_body,
        grid=(n // br,),
        in_specs=[
            pl.BlockSpec((2, br, d), lambda i: (0, i, 0)),
            pl.BlockSpec((br, d), lambda i: (i, 0)),
            pl.BlockSpec((2, br, 1), lambda i: (0, i, 0)),
            pl.BlockSpec((2, d), lambda i: (0, 0)),
            pl.BlockSpec((d, d), lambda i: (0, 0)),
        ],
        out_specs=pl.BlockSpec((br, d), lambda i: (i, 0)),
        out_shape=jax.ShapeDtypeStruct((n, d), jnp.float32),
    )(p, h, hp, sc, w)


def _out_body(p_ref, h_ref, hp_ref, sc_ref, o_ref):
    dv = _dinv_block(hp_ref)
    u = (p_ref[0] + p_ref[1] + h_ref[...]) * dv
    o_ref[...] = jnp.maximum(u * sc_ref[0:1, :] + sc_ref[1:2, :], 0.0)


def _tc_out(p, h, hp, sc):
    n, d = h.shape
    br = _pick_br(n)
    return pl.pallas_call(
        _out_body,
        grid=(n // br,),
        in_specs=[
            pl.BlockSpec((2, br, d), lambda i: (0, i, 0)),
            pl.BlockSpec((br, d), lambda i: (i, 0)),
            pl.BlockSpec((2, br, 1), lambda i: (0, i, 0)),
            pl.BlockSpec((2, d), lambda i: (0, 0)),
        ],
        out_specs=pl.BlockSpec((br, d), lambda i: (i, 0)),
        out_shape=jax.ShapeDtypeStruct((n, d), jnp.float32),
    )(p, h, hp, sc)


def kernel(x, edge_index, W1, b1, g1, bt1, m1, v1, W2, b2, g2, bt2, m2, v2):
    n, d = x.shape
    e = edge_index.shape[1]
    assert e % _CHUNK == 0
    n_chunks = e // _CHUNK
    stripe = _NS * _CHUNK
    n_pad = ((n + stripe - 1) // stripe) * stripe

    s1 = g1 * lax.rsqrt(v1 + _EPS)
    sc1 = jnp.stack([s1, (b1 - m1) * s1 + bt1])
    s2 = g2 * lax.rsqrt(v2 + _EPS)
    sc2 = jnp.stack([s2, (b2 - m2) * s2 + bt2])

    hp = _sc_degree(edge_index, n_pad, n_chunks)
    hp = hp.reshape(_NC, n_pad, 1)
    h1 = _tc_matmul_scale(x, W1, hp)
    p1 = _sc_scatter(h1, edge_index, n_pad, n_chunks)
    h2 = _tc_mid(p1, h1, hp, sc1, W2)
    p2 = _sc_scatter(h2, edge_index, n_pad, n_chunks)
    return _tc_out(p2, h2, hp, sc2)


# speedup vs baseline: 34.2166x; 1.0985x over previous
import dataclasses
import functools

import jax
import jax.numpy as jnp
from jax import lax
from jax.experimental import pallas as pl
from jax.experimental.pallas import tpu as pltpu
from jax.experimental.pallas import tpu_sc as plsc

_NC = 2
_NS = 16
_NW = _NC * _NS
_L = 16
_CHUNK = 128
_EPS = 1e-5


def _sc_mesh():
    return plsc.VectorSubcoreMesh(
        core_axis_name="c", subcore_axis_name="s",
        num_cores=_NC, num_subcores=_NS)


def _sc_degree(edge_index, n_pad, n_chunks):
    rpt = n_pad // _NS
    jmax = (n_chunks // _NW) // 2 + 2

    @functools.partial(
        pl.kernel,
        out_type=jax.ShapeDtypeStruct((_NC, n_pad), jnp.float32),
        mesh=_sc_mesh(),
        compiler_params=dataclasses.replace(pltpu.CompilerParams(),
                                            needs_layout_passes=False),
        scratch_types=[
            pltpu.VMEM((_CHUNK,), jnp.int32),
            pltpu.VMEM((_CHUNK,), jnp.int32),
            pltpu.VMEM((n_pad,), jnp.float32),
            pltpu.VMEM((_NS, rpt), jnp.float32),
            pltpu.VMEM_SHARED((_NS, n_pad), jnp.float32),
            pltpu.SemaphoreType.DMA,
            pltpu.SemaphoreType.DMA,
        ],
    )
    def k(ei_hbm, out_hbm, idx0, idx1, acc_t, red_v, shared, sem_i0, sem_i1):
        cid = lax.axis_index("c")
        sid = lax.axis_index("s")
        w = sid * _NC + cid
        k_t = (n_chunks - 1 - w) // _NW + 1
        last_c = n_chunks - 1
        ones = jnp.ones((_L,), jnp.float32)

        def idx_start(c, idx_v, sem):
            pltpu.async_copy(ei_hbm.at[1, pl.ds(c * _CHUNK, _CHUNK)],
                             idx_v, sem)

        def idx_wait(idx_v, sem):
            pltpu.make_async_copy(ei_hbm.at[1, pl.ds(0, _CHUNK)],
                                  idx_v, sem).wait()

        idx_start(w, idx0, sem_i0)

        @pl.loop(0, n_pad, step=_L)
        def _(r0):
            acc_t[pl.ds(r0, _L)] = jnp.zeros((_L,), jnp.float32)

        def item(k_idx, idx_a, sem_a, idx_b, sem_b):
            @pl.when(k_idx < k_t)
            def _():
                c_next = jnp.minimum(w + (k_idx + 1) * _NW, last_c)
                idx_start(c_next, idx_b, sem_b)
                idx_wait(idx_a, sem_a)
                for c0 in range(0, _CHUNK, _L):
                    plsc.addupdate_scatter(
                        acc_t, [idx_a[pl.ds(c0, _L)]], ones)

        item(0, idx0, sem_i0, idx1, sem_i1)
        item(1, idx1, sem_i1, idx0, sem_i0)

        @pl.loop(1, jmax)
        def _(j):
            item(2 * j, idx0, sem_i0, idx1, sem_i1)
            item(2 * j + 1, idx1, sem_i1, idx0, sem_i0)

        @pl.when(k_t % 2 == 0)
        def _():
            idx_wait(idx0, sem_i0)

        @pl.when(k_t % 2 == 1)
        def _():
            idx_wait(idx1, sem_i1)

        pltpu.sync_copy(acc_t, shared.at[sid])
        plsc.subcore_barrier()
        col0 = sid * rpt
        pltpu.sync_copy(shared.at[:, pl.ds(col0, rpt)], red_v)
        for r in range(1, _NS):
            @pl.loop(0, rpt, step=_L)
            def _(c0, _r=r):
                red_v[0, pl.ds(c0, _L)] = (red_v[0, pl.ds(c0, _L)]
                                           + red_v[_r, pl.ds(c0, _L)])
        pltpu.sync_copy(red_v.at[0], out_hbm.at[cid, pl.ds(col0, rpt)])

    return k(edge_index)


def _sc_scatter(h, edge_index, n_pad, n_chunks):
    d = h.shape[1]
    rpt = n_pad // _NS

    jmax = (n_chunks // _NW) // 2 + 2

    @functools.partial(
        pl.kernel,
        out_type=jax.ShapeDtypeStruct((_NC, n_pad, d), jnp.float32),
        mesh=_sc_mesh(),
        scratch_types=[
            pltpu.VMEM((2, _CHUNK), jnp.int32),
            pltpu.VMEM((2, _CHUNK), jnp.int32),
            pltpu.VMEM((_CHUNK,), jnp.int32),
            pltpu.VMEM((_CHUNK,), jnp.int32),
            pltpu.VMEM((_CHUNK,), jnp.int32),
            pltpu.VMEM((_CHUNK,), jnp.int32),
            pltpu.VMEM((_CHUNK, d), jnp.float32),
            pltpu.VMEM((_CHUNK, d), jnp.float32),
            pltpu.VMEM_SHARED((n_pad, d), jnp.float32),
            pltpu.SemaphoreType.DMA,
            pltpu.SemaphoreType.DMA,
            pltpu.SemaphoreType.DMA,
            pltpu.SemaphoreType.DMA,
            pltpu.SemaphoreType.DMA,
            pltpu.SemaphoreType.DMA,
        ],
    )
    def k(h_hbm, ei_hbm, out_hbm, idx0, idx1, ssh0, ssh1, dsh0, dsh1,
          rows0, rows1, acc, sem_i0, sem_i1, sem_g0, sem_g1,
          sem_s0, sem_s1):
        cid = lax.axis_index("c")
        sid = lax.axis_index("s")
        w = sid * _NC + cid
        k_t = (n_chunks - 1 - w) // _NW + 1
        last_c = n_chunks - 1

        def idx_start(c, idx_v, sem):
            pltpu.async_copy(ei_hbm.at[:, pl.ds(c * _CHUNK, _CHUNK)],
                             idx_v, sem)

        def idx_wait(idx_v, sem):
            pltpu.make_async_copy(ei_hbm.at[:, pl.ds(0, _CHUNK)],
                                  idx_v, sem).wait()

        idx_start(w, idx0, sem_i0)

        @pl.loop(0, _CHUNK)
        def _(r):
            for c0 in range(0, d, _L):
                rows0[r, pl.ds(c0, _L)] = jnp.zeros((_L,), jnp.float32)

        @pl.loop(0, rpt, step=_CHUNK)
        def _(r0):
            pltpu.sync_copy(rows0, acc.at[pl.ds(sid * rpt + r0, _CHUNK)])

        plsc.subcore_barrier()

        def item(k_idx, idx_a, sem_ia, ssh_a, dsh_a, rows_a, sem_ga, sem_sa,
                 dsh_b, rows_b, sem_gb, sem_sb, idx_b, sem_ib, first,
                 do_prev):
            @pl.when(k_idx < k_t)
            def _():
                c_next = jnp.minimum(w + (k_idx + 1) * _NW, last_c)
                idx_start(c_next, idx_b, sem_ib)
                idx_wait(idx_a, sem_ia)
                if not first:
                    pltpu.make_async_copy(rows_a, acc.at[dsh_a],
                                          sem_sa).wait()
                for c0 in range(0, _CHUNK, _L):
                    ssh_a[pl.ds(c0, _L)] = idx_a[0, pl.ds(c0, _L)]
                    dsh_a[pl.ds(c0, _L)] = idx_a[1, pl.ds(c0, _L)]
                pltpu.async_copy(h_hbm.at[ssh_a], rows_a, sem_ga)
                if do_prev:
                    pltpu.make_async_copy(h_hbm.at[ssh_a], rows_b,
                                          sem_gb).wait()
                    pltpu.async_copy(rows_b, acc.at[dsh_b], sem_sb,
                                     add=True)

        item(0, idx0, sem_i0, ssh0, dsh0, rows0, sem_g0, sem_s0,
             dsh1, rows1, sem_g1, sem_s1, idx1, sem_i1, True, False)
        item(1, idx1, sem_i1, ssh1, dsh1, rows1, sem_g1, sem_s1,
             dsh0, rows0, sem_g0, sem_s0, idx0, sem_i0, True, True)

        @pl.loop(1, jmax)
        def _(j):
            item(2 * j, idx0, sem_i0, ssh0, dsh0, rows0, sem_g0, sem_s0,
                 dsh1, rows1, sem_g1, sem_s1, idx1, sem_i1, False, True)
            item(2 * j + 1, idx1, sem_i1, ssh1, dsh1, rows1, sem_g1, sem_s1,
                 dsh0, rows0, sem_g0, sem_s0, idx0, sem_i0, False, True)

        @pl.when(k_t % 2 == 1)
        def _():
            pltpu.make_async_copy(h_hbm.at[ssh0], rows0, sem_g0).wait()
            pltpu.async_copy(rows0, acc.at[dsh0], sem_s0, add=True)
            idx_wait(idx1, sem_i1)

        @pl.when(k_t % 2 == 0)
        def _():
            pltpu.make_async_copy(h_hbm.at[ssh1], rows1, sem_g1).wait()
            pltpu.async_copy(rows1, acc.at[dsh1], sem_s1, add=True)
            idx_wait(idx0, sem_i0)

        pltpu.make_async_copy(rows0, acc.at[dsh0], sem_s0).wait()
        pltpu.make_async_copy(rows1, acc.at[dsh1], sem_s1).wait()

        plsc.subcore_barrier()
        pltpu.sync_copy(acc.at[pl.ds(sid * rpt, rpt)],
                        out_hbm.at[cid, pl.ds(sid * rpt, rpt)])

    return k(h, edge_index)


def _pick_br(n):
    for br in (2000, 1024, 1000, 512, 500, 400, 256, 200, 128, 80, 40, 16, 8):
        if n % br == 0:
            return br
    return n


def _dinv_block(hp_ref):
    deg = hp_ref[0] + hp_ref[1] + 1.0
    return lax.rsqrt(deg)


def _mm_body(x_ref, w_ref, hp_ref, o_ref):
    dv = _dinv_block(hp_ref)
    o_ref[...] = jnp.dot(x_ref[...], w_ref[...],
                         preferred_element_type=jnp.float32) * dv


def _tc_matmul_scale(x, w, hp):
    n, d = x.shape
    br = _pick_br(n)
    return pl.pallas_call(
        _mm_body,
        grid=(n // br,),
        in_specs=[
            pl.BlockSpec((br, d), lambda i: (i, 0)),
            pl.BlockSpec((d, d), lambda i: (0, 0)),
            pl.BlockSpec((2, br, 1), lambda i: (0, i, 0)),
        ],
        out_specs=pl.BlockSpec((br, d), lambda i: (i, 0)),
        out_shape=jax.ShapeDtypeStruct((n, d), jnp.float32),
    )(x, w, hp)


def _mid_body(p_ref, h_ref, hp_ref, sc_ref, w_ref, o_ref):
    dv = _dinv_block(hp_ref)
    u = (p_ref[0] + p_ref[1] + h_ref[...]) * dv
    z = jnp.maximum(u * sc_ref[0:1, :] + sc_ref[1:2, :], 0.0)
    o_ref[...] = jnp.dot(z, w_ref[...],
                         preferred_element_type=jnp.float32) * dv


def _tc_mid(p, h, hp, sc, w):
    n, d = h.shape
    br = _pick_br(n)
    return pl.pallas_call(
        _mid_body,
        grid=(n // br,),
        in_specs=[
            pl.BlockSpec((2, br, d), lambda i: (0, i, 0)),
            pl.BlockSpec((br, d), lambda i: (i, 0)),
            pl.BlockSpec((2, br, 1), lambda i: (0, i, 0)),
            pl.BlockSpec((2, d), lambda i: (0, 0)),
            pl.BlockSpec((d, d), lambda i: (0, 0)),
        ],
        out_specs=pl.BlockSpec((br, d), lambda i: (i, 0)),
        out_shape=jax.ShapeDtypeStruct((n, d), jnp.float32),
    )(p, h, hp, sc, w)


def _out_body(p_ref, h_ref, hp_ref, sc_ref, o_ref):
    dv = _dinv_block(hp_ref)
    u = (p_ref[0] + p_ref[1] + h_ref[...]) * dv
    o_ref[...] = jnp.maximum(u * sc_ref[0:1, :] + sc_ref[1:2, :], 0.0)


def _tc_out(p, h, hp, sc):
    n, d = h.shape
    br = _pick_br(n)
    return pl.pallas_call(
        _out_body,
        grid=(n // br,),
        in_specs=[
            pl.BlockSpec((2, br, d), lambda i: (0, i, 0)),
            pl.BlockSpec((br, d), lambda i: (i, 0)),
            pl.BlockSpec((2, br, 1), lambda i: (0, i, 0)),
            pl.BlockSpec((2, d), lambda i: (0, 0)),
        ],
        out_specs=pl.BlockSpec((br, d), lambda i: (i, 0)),
        out_shape=jax.ShapeDtypeStruct((n, d), jnp.float32),
    )(p, h, hp, sc)


def kernel(x, edge_index, W1, b1, g1, bt1, m1, v1, W2, b2, g2, bt2, m2, v2):
    n, d = x.shape
    e = edge_index.shape[1]
    assert e % _CHUNK == 0
    n_chunks = e // _CHUNK
    stripe = _NS * _CHUNK
    n_pad = ((n + stripe - 1) // stripe) * stripe

    s1 = g1 * lax.rsqrt(v1 + _EPS)
    sc1 = jnp.stack([s1, (b1 - m1) * s1 + bt1])
    s2 = g2 * lax.rsqrt(v2 + _EPS)
    sc2 = jnp.stack([s2, (b2 - m2) * s2 + bt2])

    hp = _sc_degree(edge_index, n_pad, n_chunks)
    hp = hp.reshape(_NC, n_pad, 1)
    h1 = _tc_matmul_scale(x, W1, hp)
    p1 = _sc_scatter(h1, edge_index, n_pad, n_chunks)
    h2 = _tc_mid(p1, h1, hp, sc1, W2)
    p2 = _sc_scatter(h2, edge_index, n_pad, n_chunks)
    return _tc_out(p2, h2, hp, sc2)
